# baseline (speedup 1.0000x reference)
import functools

import jax
import jax.numpy as jnp
from jax import lax
from jax.experimental import pallas as pl
from jax.experimental.pallas import tpu as pltpu
from jax.experimental.pallas import tpu_sc as plsc

ENTREZ = 100000
D = 128
B = 4096
H = 50
N = B * H
NC = 2
NS = 16
NW = NC * NS
CH = B // NW
NCH = H
NB = 5
NLOOP = NCH // NB


def _body(xt_hbm, table_hbm, out_hbm, idx_v, *rest):
    bufs = rest[:NB]
    sg = rest[NB:2 * NB]
    so = rest[2 * NB:3 * NB]
    wid = lax.axis_index("s") * NC + lax.axis_index("c")
    col = wid * CH

    pltpu.sync_copy(xt_hbm.at[:, pl.ds(col, CH)], idx_v)

    def g_start(c, b):
        pltpu.async_copy(table_hbm.at[idx_v.at[c]], bufs[b], sg[b])

    def g_wait(b):
        pltpu.make_async_copy(table_hbm.at[idx_v.at[0]], bufs[b], sg[b]).wait()

    def o_start(c, b):
        pltpu.async_copy(bufs[b], out_hbm.at[pl.ds(c * B + col, CH)], so[b])

    def o_wait(b):
        pltpu.make_async_copy(bufs[b], out_hbm.at[pl.ds(col, CH)], so[b]).wait()

    for b in range(NB - 1):
        g_start(b, b)

    def loop(i, carry):
        for b in range(NB):
            c = NB * i + b
            g_wait(b)
            o_start(c, b)
            cr = c + NB - 1

            @pl.when(cr < NCH)
            def _():
                @pl.when(c >= 1)
                def _():
                    o_wait((b + NB - 1) % NB)

                g_start(cr, (b + NB - 1) % NB)

        return carry

    lax.fori_loop(0, NLOOP, loop, 0)
    for b in range(NB):
        o_wait(b)


def kernel(x, table):
    xt = x.T.astype(jnp.int32)
    mesh = plsc.VectorSubcoreMesh(core_axis_name="c", subcore_axis_name="s")
    run = functools.partial(
        pl.kernel,
        mesh=mesh,
        out_type=jax.ShapeDtypeStruct((N, D), jnp.float32),
        scratch_types=(
            [pltpu.VMEM((NCH, CH), jnp.int32)]
            + [pltpu.VMEM((CH, D), jnp.float32) for _ in range(NB)]
            + [pltpu.SemaphoreType.DMA for _ in range(2 * NB)]
        ),
    )(_body)
    out = run(xt, table)
    return out.reshape(H, B, D).transpose(1, 0, 2)


# speedup vs baseline: 1.0015x; 1.0015x over previous
import functools

import jax
import jax.numpy as jnp
from jax import lax
from jax.experimental import pallas as pl
from jax.experimental.pallas import tpu as pltpu
from jax.experimental.pallas import tpu_sc as plsc

ENTREZ = 100000
D = 128
B = 4096
H = 50
N = B * H
NC = 2
NS = 16
NW = NC * NS
CH = B // NW
NCH = H
NB = 5
NLOOP = NCH // NB


def _body(xt_hbm, table_hbm, out_hbm, idx_v, *rest):
    bufs = rest[:NB]
    sg = rest[NB:2 * NB]
    so = rest[2 * NB:3 * NB]
    st = rest[3 * NB]
    wid = lax.axis_index("s") * NC + lax.axis_index("c")
    col = wid * CH

    head = 8
    pltpu.sync_copy(xt_hbm.at[pl.ds(0, head), pl.ds(col, CH)],
                    idx_v.at[pl.ds(0, head)])
    tail = pltpu.async_copy(xt_hbm.at[pl.ds(head, NCH - head), pl.ds(col, CH)],
                            idx_v.at[pl.ds(head, NCH - head)], st)

    def g_start(c, b):
        pltpu.async_copy(table_hbm.at[idx_v.at[c]], bufs[b], sg[b])

    def g_wait(b):
        pltpu.make_async_copy(table_hbm.at[idx_v.at[0]], bufs[b], sg[b]).wait()

    def o_start(c, b):
        pltpu.async_copy(bufs[b], out_hbm.at[pl.ds(c * B + col, CH)], so[b])

    def o_wait(b):
        pltpu.make_async_copy(bufs[b], out_hbm.at[pl.ds(col, CH)], so[b]).wait()

    for b in range(NB - 1):
        g_start(b, b)
    tail.wait()

    def loop(i, carry):
        for b in range(NB):
            c = NB * i + b
            g_wait(b)
            o_start(c, b)
            cr = c + NB - 1

            @pl.when(cr < NCH)
            def _():
                @pl.when(c >= 1)
                def _():
                    o_wait((b + NB - 1) % NB)

                g_start(cr, (b + NB - 1) % NB)

        return carry

    lax.fori_loop(0, NLOOP, loop, 0)
    for b in range(NB):
        o_wait(b)


def kernel(x, table):
    xt = x.T.astype(jnp.int32)
    mesh = plsc.VectorSubcoreMesh(core_axis_name="c", subcore_axis_name="s")
    run = functools.partial(
        pl.kernel,
        mesh=mesh,
        out_type=jax.ShapeDtypeStruct((N, D), jnp.float32),
        scratch_types=(
            [pltpu.VMEM((NCH, CH), jnp.int32)]
            + [pltpu.VMEM((CH, D), jnp.float32) for _ in range(NB)]
            + [pltpu.SemaphoreType.DMA for _ in range(2 * NB + 1)]
        ),
    )(_body)
    out = run(xt, table)
    return out.reshape(H, B, D).transpose(1, 0, 2)
